# Initial kernel scaffold; baseline (speedup 1.0000x reference)
import jax
import jax.numpy as jnp
from jax.experimental import pallas as pl


def kernel(cur_epoch, users, pos_items, neg_items, adj_rows, adj_cols, adj_vals, user_embed, item_embed, W_user_gate, b_user_gate, W_item_gate, b_item_gate, W_pos_gate, b_pos_gate, W_neg_gate, b_neg_gate):
    raise NotImplementedError("write your pallas kernel here")


# speedup vs baseline: 1.6368x; 1.6368x over previous
import functools

import jax
import jax.numpy as jnp
from jax import lax
from jax.experimental import pallas as pl
from jax.experimental.pallas import tpu as pltpu
from jax.experimental.pallas import tpu_sc as plsc

N_USERS = 10000
N_ITEMS = 40000
N_NODES = 50000
D = 64
NNZ = 800000
BATCH = 4096
NEGS = 16
WARMUP = 100.0
DECAY = 1e-4

NC = 2
NS = 16
NW = NC * NS
LANE = 16

HALF = N_NODES // NC
TILE_ROWS = 1568
ACC_ROWS = NS * TILE_ROWS
DUMMY_ROW = ACC_ROWS - 8
LAST_ROWS = HALF - (NS - 1) * TILE_ROWS
EPT = 51200
NNZ_PAD = NS * EPT
CHUNK = 256
NCHUNK = EPT // CHUNK
NSTREAM = CHUNK // 128


def _hop_body(table, rows, cols, vals, out, acc, cvm, rvm, vvm, lvm, gvm, sem):
    core = lax.axis_index("c")
    sid = lax.axis_index("s")
    row_base = core * HALF

    def _zero_row(i, carry):
        for d4 in range(D // LANE):
            gvm[i, pl.ds(d4 * LANE, LANE)] = jnp.zeros((LANE,), jnp.float32)
        return carry

    lax.fori_loop(0, CHUNK, _zero_row, 0)
    abase = sid * TILE_ROWS
    for t in range(TILE_ROWS // CHUNK):
        pltpu.sync_copy(gvm, acc.at[pl.ds(abase + t * CHUNK, CHUNK)])
    _zrem = TILE_ROWS % CHUNK
    if _zrem:
        pltpu.sync_copy(gvm.at[pl.ds(0, _zrem)],
                        acc.at[pl.ds(abase + TILE_ROWS - _zrem, _zrem)])
    plsc.subcore_barrier()

    def _chunk(ch, carry):
        ebase = sid * EPT + ch * CHUNK
        pltpu.sync_copy(cols.at[pl.ds(ebase, CHUNK)], cvm)
        pltpu.sync_copy(rows.at[pl.ds(ebase, CHUNK)], rvm)
        pltpu.sync_copy(vals.at[pl.ds(ebase, CHUNK)], vvm)
        for j in range(NSTREAM):
            pltpu.async_copy(table.at[cvm.at[pl.ds(j * 128, 128)]],
                             gvm.at[pl.ds(j * 128, 128)], sem).wait()
        for j in range(NSTREAM):
            for g in range(128 // LANE):
                r = rvm[pl.ds(j * 128 + g * LANE, LANE)]
                loc = r - row_base
                ok = (loc >= 0) & (loc < HALF)
                lvm[j, pl.ds(g * LANE, LANE)] = jnp.where(ok, loc, DUMMY_ROW)
        def _scale(g, carry2):
            vv = vvm[pl.ds(g * LANE, LANE)]
            for k in range(LANE):
                v = vv[k]
                e = g * LANE + k
                for d4 in range(D // LANE):
                    sl = pl.ds(d4 * LANE, LANE)
                    gvm[e, sl] = gvm[e, sl] * v
            return carry2

        lax.fori_loop(0, CHUNK // LANE, _scale, 0)
        for j in range(NSTREAM):
            pltpu.sync_copy(gvm.at[pl.ds(j * 128, 128)],
                            acc.at[lvm.at[j]], add=True)
        return carry

    lax.fori_loop(0, NCHUNK, _chunk, 0)
    plsc.subcore_barrier()

    out_base = row_base + sid * TILE_ROWS
    pltpu.sync_copy(acc.at[pl.ds(abase, LAST_ROWS)],
                    out.at[pl.ds(out_base, LAST_ROWS)])

    @pl.when(sid < NS - 1)
    def _():
        pltpu.sync_copy(acc.at[pl.ds(abase + LAST_ROWS, TILE_ROWS - LAST_ROWS)],
                        out.at[pl.ds(out_base + LAST_ROWS, TILE_ROWS - LAST_ROWS)])


@functools.cache
def _get_hop_call():
    return pl.kernel(
        _hop_body,
        out_type=jax.ShapeDtypeStruct((N_NODES, D), jnp.float32),
        mesh=plsc.VectorSubcoreMesh(core_axis_name="c", subcore_axis_name="s",
                                    num_cores=NC, num_subcores=NS),
        compiler_params=pltpu.CompilerParams(use_tc_tiling_on_sc=False),
        scratch_types=[
            pltpu.VMEM_SHARED((ACC_ROWS, D), jnp.float32),
            pltpu.VMEM((CHUNK,), jnp.int32),
            pltpu.VMEM((CHUNK,), jnp.int32),
            pltpu.VMEM((CHUNK,), jnp.float32),
            pltpu.VMEM((NSTREAM, 128), jnp.int32),
            pltpu.VMEM((CHUNK, D), jnp.float32),
            pltpu.SemaphoreType.DMA,
        ],
    )

U_PER_W = BATCH // NW
IDX_WORDS = 2 * U_PER_W + NEGS * U_PER_W


def _gather_body(e0, e1, e2, e3, users, pos, negs_t, s_out, p_out, n_out,
                 ivm, gvm, sem):
    core = lax.axis_index("c")
    sid = lax.axis_index("s")
    wid = sid * NC + core
    ub = wid * U_PER_W
    pltpu.sync_copy(users.at[pl.ds(ub, U_PER_W)], ivm.at[pl.ds(0, U_PER_W)])
    pltpu.sync_copy(pos.at[pl.ds(ub, U_PER_W)], ivm.at[pl.ds(U_PER_W, U_PER_W)])
    for j in range(NEGS):
        pltpu.sync_copy(negs_t.at[pl.ds(j * BATCH + ub, U_PER_W)],
                        ivm.at[pl.ds((2 + j) * U_PER_W, U_PER_W)])
    for g in range(U_PER_W // LANE, IDX_WORDS // LANE):
        sl = pl.ds(g * LANE, LANE)
        ivm[sl] = ivm[sl] + N_USERS
    for l, t in enumerate((e0, e1, e2, e3)):
        pltpu.async_copy(t.at[ivm.at[pl.ds(0, U_PER_W)]], gvm, sem).wait()
        pltpu.sync_copy(gvm, s_out.at[l, pl.ds(ub, U_PER_W)])
        pltpu.async_copy(t.at[ivm.at[pl.ds(U_PER_W, U_PER_W)]], gvm, sem).wait()
        pltpu.sync_copy(gvm, p_out.at[l, pl.ds(ub, U_PER_W)])
        for j in range(NEGS):
            pltpu.async_copy(t.at[ivm.at[pl.ds((2 + j) * U_PER_W, U_PER_W)]],
                             gvm, sem).wait()
            pltpu.sync_copy(gvm, n_out.at[l, j, pl.ds(ub, U_PER_W)])


@functools.cache
def _get_gather_call():
    return pl.kernel(
        _gather_body,
        out_type=(
            jax.ShapeDtypeStruct((4, BATCH, D), jnp.float32),
            jax.ShapeDtypeStruct((4, BATCH, D), jnp.float32),
            jax.ShapeDtypeStruct((4, NEGS, BATCH, D), jnp.float32),
        ),
        mesh=plsc.VectorSubcoreMesh(core_axis_name="c", subcore_axis_name="s",
                                    num_cores=NC, num_subcores=NS),
        compiler_params=pltpu.CompilerParams(use_tc_tiling_on_sc=False),
        scratch_types=[
            pltpu.VMEM((IDX_WORDS,), jnp.int32),
            pltpu.VMEM((U_PER_W, D), jnp.float32),
            pltpu.SemaphoreType.DMA,
        ],
    )

BB = 256
GB = BATCH // BB


def _dotT(x, w):
    return lax.dot_general(x, w, (((1,), (1,)), ((), ())),
                           preferred_element_type=jnp.float32)


def _loss_body(factor_ref, s_ref, p_ref, n_ref, wu, bu, wi, bi, wp, bp,
               wn, bn, loss_ref, reg_ref):
    factor = factor_ref[0, 0]
    u_acc = jnp.zeros((BB, D), jnp.float32)
    pos_acc = jnp.zeros((BB, D), jnp.float32)
    neg_acc = jnp.zeros((BB, D), jnp.float32)
    sel0 = jnp.zeros((BB, D), jnp.float32)
    for l in range(4):
        s_l = s_ref[l]
        p_l = p_ref[l]
        gate_p = jax.nn.sigmoid(_dotT(p_l, wi[...]) + bi[...]
                                + _dotT(s_l, wu[...]) + bu[...])
        gated_p = p_l * gate_p
        gp = _dotT(gated_p, wp[...]) + bp[...]
        best = jnp.full((BB, 1), -1e30, jnp.float32)
        bidx = jnp.zeros((BB, 1), jnp.int32)
        for j in range(NEGS):
            n_j = n_ref[l, j]
            gate_n = jax.nn.sigmoid(_dotT(n_j, wn[...]) + bn[...] + gp)
            n_sel = factor * n_j - n_j * gate_n
            sc = jnp.sum(n_sel * s_l, axis=1, keepdims=True)
            upd = sc > best
            bidx = jnp.where(upd, j, bidx)
            best = jnp.where(upd, sc, best)
        sel = jnp.zeros((BB, D), jnp.float32)
        for j in range(NEGS):
            sel = sel + jnp.where(bidx == j, n_ref[l, j], 0.0)
        u_acc = u_acc + s_l
        pos_acc = pos_acc + p_l
        neg_acc = neg_acc + sel
        if l == 0:
            sel0 = sel
            reg_blk = (jnp.sum(s_l * s_l) + jnp.sum(p_l * p_l))
    reg_blk = reg_blk + jnp.sum(sel0 * sel0)
    u_e = u_acc * 0.25
    pos_e = pos_acc * 0.25
    neg_e = neg_acc * 0.25
    d_sc = jnp.sum(u_e * neg_e, axis=1) - jnp.sum(u_e * pos_e, axis=1)
    blk_loss = jnp.sum(jnp.log(1.0 + jnp.exp(d_sc)))

    @pl.when(pl.program_id(0) == 0)
    def _():
        loss_ref[0, 0] = 0.0
        reg_ref[0, 0] = 0.0

    loss_ref[0, 0] += blk_loss
    reg_ref[0, 0] += reg_blk


_loss_call = pl.pallas_call(
    _loss_body,
    grid=(GB,),
    in_specs=[
        pl.BlockSpec(memory_space=pltpu.SMEM),
        pl.BlockSpec((4, BB, D), lambda i: (0, i, 0)),
        pl.BlockSpec((4, BB, D), lambda i: (0, i, 0)),
        pl.BlockSpec((4, NEGS, BB, D), lambda i: (0, 0, i, 0)),
        pl.BlockSpec((D, D), lambda i: (0, 0)),
        pl.BlockSpec((1, D), lambda i: (0, 0)),
        pl.BlockSpec((D, D), lambda i: (0, 0)),
        pl.BlockSpec((1, D), lambda i: (0, 0)),
        pl.BlockSpec((D, D), lambda i: (0, 0)),
        pl.BlockSpec((1, D), lambda i: (0, 0)),
        pl.BlockSpec((D, D), lambda i: (0, 0)),
        pl.BlockSpec((1, D), lambda i: (0, 0)),
    ],
    out_specs=[
        pl.BlockSpec(memory_space=pltpu.SMEM),
        pl.BlockSpec(memory_space=pltpu.SMEM),
    ],
    out_shape=[
        jax.ShapeDtypeStruct((1, 1), jnp.float32),
        jax.ShapeDtypeStruct((1, 1), jnp.float32),
    ],
)


def kernel(cur_epoch, users, pos_items, neg_items, adj_rows, adj_cols,
           adj_vals, user_embed, item_embed,
           W_user_gate, b_user_gate, W_item_gate, b_item_gate,
           W_pos_gate, b_pos_gate, W_neg_gate, b_neg_gate):
    pad = NNZ_PAD - NNZ
    rows_p = jnp.concatenate([adj_rows, jnp.zeros((pad,), jnp.int32)])
    cols_p = jnp.concatenate([adj_cols, jnp.zeros((pad,), jnp.int32)])
    vals_p = jnp.concatenate([adj_vals, jnp.zeros((pad,), jnp.float32)])
    e0 = jnp.concatenate([user_embed, item_embed], axis=0)
    hop = _get_hop_call()
    e1 = hop(e0, rows_p, cols_p, vals_p)
    e2 = hop(e1, rows_p, cols_p, vals_p)
    e3 = hop(e2, rows_p, cols_p, vals_p)
    negs_t = neg_items.T.reshape(-1)
    s_all, p_all, n_all = _get_gather_call()(e0, e1, e2, e3, users,
                                             pos_items, negs_t)
    factor = (1.0 - jnp.minimum(
        1.0, jnp.asarray(cur_epoch).astype(jnp.float32) / WARMUP)).reshape(1, 1)
    loss_sum, reg_sum = _loss_call(
        factor, s_all, p_all, n_all,
        W_user_gate, b_user_gate.reshape(1, D),
        W_item_gate, b_item_gate.reshape(1, D),
        W_pos_gate, b_pos_gate.reshape(1, D),
        W_neg_gate, b_neg_gate.reshape(1, D))
    mf_loss = loss_sum[0, 0] / BATCH
    emb_loss = (DECAY / (2.0 * BATCH)) * reg_sum[0, 0]
    return mf_loss + emb_loss, mf_loss, emb_loss


# speedup vs baseline: 2.0158x; 1.2315x over previous
import functools

import jax
import jax.numpy as jnp
from jax import lax
from jax.experimental import pallas as pl
from jax.experimental.pallas import tpu as pltpu
from jax.experimental.pallas import tpu_sc as plsc

N_USERS = 10000
N_ITEMS = 40000
N_NODES = 50000
D = 64
NNZ = 800000
BATCH = 4096
NEGS = 16
WARMUP = 100.0
DECAY = 1e-4

NC = 2
NS = 16
NW = NC * NS
LANE = 16

HALF = N_NODES // NC
TILE_ROWS = 1568
ACC_ROWS = NS * TILE_ROWS
DUMMY_ROW = ACC_ROWS - 8
LAST_ROWS = HALF - (NS - 1) * TILE_ROWS
EPT = 51200
NNZ_PAD = NS * EPT
CHUNK = 128
NSTEP = EPT // CHUNK


def _hop_body(table, rows, cols, vals, out, acc,
              cvm, rvm, vvm, lvm, gvm, gsem0, gsem1, msem0, msem1,
              ssem0, ssem1, sem):
    core = lax.axis_index("c")
    sid = lax.axis_index("s")
    row_base = core * HALF
    gsem = (gsem0, gsem1)
    msem = (msem0, msem1)
    ssem = (ssem0, ssem1)

    def _zero_row(i, carry):
        for d4 in range(D // LANE):
            gvm[0, i, pl.ds(d4 * LANE, LANE)] = jnp.zeros((LANE,), jnp.float32)
        return carry

    lax.fori_loop(0, CHUNK, _zero_row, 0)
    abase = sid * TILE_ROWS
    for t in range(TILE_ROWS // CHUNK):
        pltpu.sync_copy(gvm.at[0], acc.at[pl.ds(abase + t * CHUNK, CHUNK)])
    _zrem = TILE_ROWS % CHUNK
    if _zrem:
        pltpu.sync_copy(gvm.at[0, pl.ds(0, _zrem)],
                        acc.at[pl.ds(abase + TILE_ROWS - _zrem, _zrem)])
    plsc.subcore_barrier()

    ebase0 = sid * EPT

    def _meta_start(t, b):
        off = ebase0 + t * CHUNK
        pltpu.async_copy(cols.at[pl.ds(off, CHUNK)], cvm.at[b], msem[b])
        pltpu.async_copy(rows.at[pl.ds(off, CHUNK)], rvm.at[b], msem[b])
        pltpu.async_copy(vals.at[pl.ds(off, CHUNK)], vvm.at[b], msem[b])

    def _meta_wait(b):
        pltpu.make_async_copy(cols.at[pl.ds(0, CHUNK)], cvm.at[b], msem[b]).wait()
        pltpu.make_async_copy(rows.at[pl.ds(0, CHUNK)], rvm.at[b], msem[b]).wait()
        pltpu.make_async_copy(vals.at[pl.ds(0, CHUNK)], vvm.at[b], msem[b]).wait()

    def _gather_start(b):
        pltpu.async_copy(table.at[cvm.at[b]], gvm.at[b], gsem[b])

    def _gather_wait(b):
        pltpu.make_async_copy(table.at[cvm.at[b]], gvm.at[b], gsem[b]).wait()

    def _scatter_start(b):
        pltpu.async_copy(gvm.at[b], acc.at[lvm.at[b]], ssem[b], add=True)

    def _scatter_wait(b):
        pltpu.make_async_copy(gvm.at[b], acc.at[lvm.at[b]], ssem[b]).wait()

    _meta_start(0, 0)
    _meta_wait(0)
    _gather_start(0)
    _meta_start(1, 1)

    def _step(t, b):
        nb = 1 - b
        _gather_wait(b)
        for g in range(CHUNK // LANE):
            r = rvm[b, pl.ds(g * LANE, LANE)]
            loc = r - row_base
            ok = (loc >= 0) & (loc < HALF)
            lvm[b, pl.ds(g * LANE, LANE)] = jnp.where(ok, loc, DUMMY_ROW)

        def _scale(g, carry2):
            vv = vvm[b, pl.ds(g * LANE, LANE)]
            for k in range(LANE):
                v = vv[k]
                e = g * LANE + k
                for d4 in range(D // LANE):
                    sl = pl.ds(d4 * LANE, LANE)
                    gvm[b, e, sl] = gvm[b, e, sl] * v
            return carry2

        lax.fori_loop(0, CHUNK // LANE, _scale, 0)

        @pl.when(t + 2 < NSTEP)
        def _():
            _meta_start(t + 2, b)

        _scatter_start(b)

        @pl.when((t >= 1) & (t + 1 < NSTEP))
        def _():
            _scatter_wait(nb)

        @pl.when(t + 1 < NSTEP)
        def _():
            _meta_wait(nb)
            _gather_start(nb)

    def _step2(i2, carry):
        _step(2 * i2, 0)
        _step(2 * i2 + 1, 1)
        return carry

    lax.fori_loop(0, NSTEP // 2, _step2, 0)
    _scatter_wait(0)
    _scatter_wait(1)
    plsc.subcore_barrier()

    out_base = row_base + sid * TILE_ROWS
    pltpu.sync_copy(acc.at[pl.ds(abase, LAST_ROWS)],
                    out.at[pl.ds(out_base, LAST_ROWS)])

    @pl.when(sid < NS - 1)
    def _():
        pltpu.sync_copy(acc.at[pl.ds(abase + LAST_ROWS, TILE_ROWS - LAST_ROWS)],
                        out.at[pl.ds(out_base + LAST_ROWS, TILE_ROWS - LAST_ROWS)])


@functools.cache
def _get_hop_call():
    return pl.kernel(
        _hop_body,
        out_type=jax.ShapeDtypeStruct((N_NODES, D), jnp.float32),
        mesh=plsc.VectorSubcoreMesh(core_axis_name="c", subcore_axis_name="s",
                                    num_cores=NC, num_subcores=NS),
        compiler_params=pltpu.CompilerParams(use_tc_tiling_on_sc=False),
        scratch_types=[
            pltpu.VMEM_SHARED((ACC_ROWS, D), jnp.float32),
            pltpu.VMEM((2, CHUNK), jnp.int32),
            pltpu.VMEM((2, CHUNK), jnp.int32),
            pltpu.VMEM((2, CHUNK), jnp.float32),
            pltpu.VMEM((2, CHUNK), jnp.int32),
            pltpu.VMEM((2, CHUNK, D), jnp.float32),
            pltpu.SemaphoreType.DMA,
            pltpu.SemaphoreType.DMA,
            pltpu.SemaphoreType.DMA,
            pltpu.SemaphoreType.DMA,
            pltpu.SemaphoreType.DMA,
            pltpu.SemaphoreType.DMA,
            pltpu.SemaphoreType.DMA,
        ],
    )

U_PER_W = BATCH // NW
IDX_WORDS = 2 * U_PER_W + NEGS * U_PER_W


def _gather_body(e0, e1, e2, e3, users, pos, negs_t, s_out, p_out, n_out,
                 ivm, gvm, sem):
    core = lax.axis_index("c")
    sid = lax.axis_index("s")
    wid = sid * NC + core
    ub = wid * U_PER_W
    pltpu.sync_copy(users.at[pl.ds(ub, U_PER_W)], ivm.at[pl.ds(0, U_PER_W)])
    pltpu.sync_copy(pos.at[pl.ds(ub, U_PER_W)], ivm.at[pl.ds(U_PER_W, U_PER_W)])
    for j in range(NEGS):
        pltpu.sync_copy(negs_t.at[pl.ds(j * BATCH + ub, U_PER_W)],
                        ivm.at[pl.ds((2 + j) * U_PER_W, U_PER_W)])
    for g in range(U_PER_W // LANE, IDX_WORDS // LANE):
        sl = pl.ds(g * LANE, LANE)
        ivm[sl] = ivm[sl] + N_USERS
    for l, t in enumerate((e0, e1, e2, e3)):
        pltpu.async_copy(t.at[ivm.at[pl.ds(0, U_PER_W)]], gvm, sem).wait()
        pltpu.sync_copy(gvm, s_out.at[l, pl.ds(ub, U_PER_W)])
        pltpu.async_copy(t.at[ivm.at[pl.ds(U_PER_W, U_PER_W)]], gvm, sem).wait()
        pltpu.sync_copy(gvm, p_out.at[l, pl.ds(ub, U_PER_W)])
        for j in range(NEGS):
            pltpu.async_copy(t.at[ivm.at[pl.ds((2 + j) * U_PER_W, U_PER_W)]],
                             gvm, sem).wait()
            pltpu.sync_copy(gvm, n_out.at[l, j, pl.ds(ub, U_PER_W)])


@functools.cache
def _get_gather_call():
    return pl.kernel(
        _gather_body,
        out_type=(
            jax.ShapeDtypeStruct((4, BATCH, D), jnp.float32),
            jax.ShapeDtypeStruct((4, BATCH, D), jnp.float32),
            jax.ShapeDtypeStruct((4, NEGS, BATCH, D), jnp.float32),
        ),
        mesh=plsc.VectorSubcoreMesh(core_axis_name="c", subcore_axis_name="s",
                                    num_cores=NC, num_subcores=NS),
        compiler_params=pltpu.CompilerParams(use_tc_tiling_on_sc=False),
        scratch_types=[
            pltpu.VMEM((IDX_WORDS,), jnp.int32),
            pltpu.VMEM((U_PER_W, D), jnp.float32),
            pltpu.SemaphoreType.DMA,
        ],
    )

BB = 256
GB = BATCH // BB


def _dotT(x, w):
    return lax.dot_general(x, w, (((1,), (1,)), ((), ())),
                           preferred_element_type=jnp.float32)


def _loss_body(factor_ref, s_ref, p_ref, n_ref, wu, bu, wi, bi, wp, bp,
               wn, bn, loss_ref, reg_ref):
    factor = factor_ref[0, 0]
    u_acc = jnp.zeros((BB, D), jnp.float32)
    pos_acc = jnp.zeros((BB, D), jnp.float32)
    neg_acc = jnp.zeros((BB, D), jnp.float32)
    sel0 = jnp.zeros((BB, D), jnp.float32)
    for l in range(4):
        s_l = s_ref[l]
        p_l = p_ref[l]
        gate_p = jax.nn.sigmoid(_dotT(p_l, wi[...]) + bi[...]
                                + _dotT(s_l, wu[...]) + bu[...])
        gated_p = p_l * gate_p
        gp = _dotT(gated_p, wp[...]) + bp[...]
        best = jnp.full((BB, 1), -1e30, jnp.float32)
        bidx = jnp.zeros((BB, 1), jnp.int32)
        for j in range(NEGS):
            n_j = n_ref[l, j]
            gate_n = jax.nn.sigmoid(_dotT(n_j, wn[...]) + bn[...] + gp)
            n_sel = factor * n_j - n_j * gate_n
            sc = jnp.sum(n_sel * s_l, axis=1, keepdims=True)
            upd = sc > best
            bidx = jnp.where(upd, j, bidx)
            best = jnp.where(upd, sc, best)
        sel = jnp.zeros((BB, D), jnp.float32)
        for j in range(NEGS):
            sel = sel + jnp.where(bidx == j, n_ref[l, j], 0.0)
        u_acc = u_acc + s_l
        pos_acc = pos_acc + p_l
        neg_acc = neg_acc + sel
        if l == 0:
            sel0 = sel
            reg_blk = (jnp.sum(s_l * s_l) + jnp.sum(p_l * p_l))
    reg_blk = reg_blk + jnp.sum(sel0 * sel0)
    u_e = u_acc * 0.25
    pos_e = pos_acc * 0.25
    neg_e = neg_acc * 0.25
    d_sc = jnp.sum(u_e * neg_e, axis=1) - jnp.sum(u_e * pos_e, axis=1)
    blk_loss = jnp.sum(jnp.log(1.0 + jnp.exp(d_sc)))

    @pl.when(pl.program_id(0) == 0)
    def _():
        loss_ref[0, 0] = 0.0
        reg_ref[0, 0] = 0.0

    loss_ref[0, 0] += blk_loss
    reg_ref[0, 0] += reg_blk


_loss_call = pl.pallas_call(
    _loss_body,
    grid=(GB,),
    in_specs=[
        pl.BlockSpec(memory_space=pltpu.SMEM),
        pl.BlockSpec((4, BB, D), lambda i: (0, i, 0)),
        pl.BlockSpec((4, BB, D), lambda i: (0, i, 0)),
        pl.BlockSpec((4, NEGS, BB, D), lambda i: (0, 0, i, 0)),
        pl.BlockSpec((D, D), lambda i: (0, 0)),
        pl.BlockSpec((1, D), lambda i: (0, 0)),
        pl.BlockSpec((D, D), lambda i: (0, 0)),
        pl.BlockSpec((1, D), lambda i: (0, 0)),
        pl.BlockSpec((D, D), lambda i: (0, 0)),
        pl.BlockSpec((1, D), lambda i: (0, 0)),
        pl.BlockSpec((D, D), lambda i: (0, 0)),
        pl.BlockSpec((1, D), lambda i: (0, 0)),
    ],
    out_specs=[
        pl.BlockSpec(memory_space=pltpu.SMEM),
        pl.BlockSpec(memory_space=pltpu.SMEM),
    ],
    out_shape=[
        jax.ShapeDtypeStruct((1, 1), jnp.float32),
        jax.ShapeDtypeStruct((1, 1), jnp.float32),
    ],
)


def kernel(cur_epoch, users, pos_items, neg_items, adj_rows, adj_cols,
           adj_vals, user_embed, item_embed,
           W_user_gate, b_user_gate, W_item_gate, b_item_gate,
           W_pos_gate, b_pos_gate, W_neg_gate, b_neg_gate):
    pad = NNZ_PAD - NNZ
    rows_p = jnp.concatenate([adj_rows, jnp.zeros((pad,), jnp.int32)])
    cols_p = jnp.concatenate([adj_cols, jnp.zeros((pad,), jnp.int32)])
    vals_p = jnp.concatenate([adj_vals, jnp.zeros((pad,), jnp.float32)])
    e0 = jnp.concatenate([user_embed, item_embed], axis=0)
    hop = _get_hop_call()
    e1 = hop(e0, rows_p, cols_p, vals_p)
    e2 = hop(e1, rows_p, cols_p, vals_p)
    e3 = hop(e2, rows_p, cols_p, vals_p)
    negs_t = neg_items.T.reshape(-1)
    s_all, p_all, n_all = _get_gather_call()(e0, e1, e2, e3, users,
                                             pos_items, negs_t)
    factor = (1.0 - jnp.minimum(
        1.0, jnp.asarray(cur_epoch).astype(jnp.float32) / WARMUP)).reshape(1, 1)
    loss_sum, reg_sum = _loss_call(
        factor, s_all, p_all, n_all,
        W_user_gate, b_user_gate.reshape(1, D),
        W_item_gate, b_item_gate.reshape(1, D),
        W_pos_gate, b_pos_gate.reshape(1, D),
        W_neg_gate, b_neg_gate.reshape(1, D))
    mf_loss = loss_sum[0, 0] / BATCH
    emb_loss = (DECAY / (2.0 * BATCH)) * reg_sum[0, 0]
    return mf_loss + emb_loss, mf_loss, emb_loss


# speedup vs baseline: 2.3037x; 1.1429x over previous
import functools

import jax
import jax.numpy as jnp
from jax import lax
from jax.experimental import pallas as pl
from jax.experimental.pallas import tpu as pltpu
from jax.experimental.pallas import tpu_sc as plsc

N_USERS = 10000
N_ITEMS = 40000
N_NODES = 50000
D = 64
NNZ = 800000
BATCH = 4096
NEGS = 16
WARMUP = 100.0
DECAY = 1e-4

NC = 2
NS = 16
NW = NC * NS
LANE = 16

HALF = N_NODES // NC
TILE_ROWS = 1568
ACC_ROWS = NS * TILE_ROWS
DUMMY_ROW = ACC_ROWS - 8
LAST_ROWS = HALF - (NS - 1) * TILE_ROWS
EPT = 51200
NNZ_PAD = NS * EPT
CHUNK = 128
NSTEP = EPT // CHUNK


def _hop_body(table, rows, cols, vals, out, acc,
              cvm, rvm, vvm, lvm, gvm, gsem0, gsem1, msem0, msem1,
              ssem0, ssem1, sem):
    core = lax.axis_index("c")
    sid = lax.axis_index("s")
    row_base = core * HALF
    gsem = (gsem0, gsem1)
    msem = (msem0, msem1)
    ssem = (ssem0, ssem1)

    def _zero_row(i, carry):
        for d4 in range(D // LANE):
            gvm[0, i, pl.ds(d4 * LANE, LANE)] = jnp.zeros((LANE,), jnp.float32)
        return carry

    lax.fori_loop(0, CHUNK, _zero_row, 0)
    abase = sid * TILE_ROWS
    for t in range(TILE_ROWS // CHUNK):
        pltpu.sync_copy(gvm.at[0], acc.at[pl.ds(abase + t * CHUNK, CHUNK)])
    _zrem = TILE_ROWS % CHUNK
    if _zrem:
        pltpu.sync_copy(gvm.at[0, pl.ds(0, _zrem)],
                        acc.at[pl.ds(abase + TILE_ROWS - _zrem, _zrem)])
    plsc.subcore_barrier()

    ebase0 = sid * EPT

    def _meta_start(t, b):
        off = ebase0 + t * CHUNK
        pltpu.async_copy(cols.at[pl.ds(off, CHUNK)], cvm.at[b], msem[b])
        pltpu.async_copy(rows.at[pl.ds(off, CHUNK)], rvm.at[b], msem[b])
        pltpu.async_copy(vals.at[pl.ds(off, CHUNK)], vvm.at[b], msem[b])

    def _meta_wait(b):
        pltpu.make_async_copy(cols.at[pl.ds(0, CHUNK)], cvm.at[b], msem[b]).wait()
        pltpu.make_async_copy(rows.at[pl.ds(0, CHUNK)], rvm.at[b], msem[b]).wait()
        pltpu.make_async_copy(vals.at[pl.ds(0, CHUNK)], vvm.at[b], msem[b]).wait()

    def _gather_start(b):
        pltpu.async_copy(table.at[cvm.at[b]], gvm.at[b], gsem[b])

    def _gather_wait(b):
        pltpu.make_async_copy(table.at[cvm.at[b]], gvm.at[b], gsem[b]).wait()

    def _scatter_start(b):
        pltpu.async_copy(gvm.at[b], acc.at[lvm.at[b]], ssem[b], add=True)

    def _scatter_wait(b):
        pltpu.make_async_copy(gvm.at[b], acc.at[lvm.at[b]], ssem[b]).wait()

    _meta_start(0, 0)
    _meta_wait(0)
    _gather_start(0)
    _meta_start(1, 1)

    def _step(t, b):
        nb = 1 - b
        _gather_wait(b)

        @pl.when((t >= 1) & (t + 1 < NSTEP))
        def _():
            _scatter_wait(nb)

        @pl.when(t + 1 < NSTEP)
        def _():
            _meta_wait(nb)
            _gather_start(nb)

        for g in range(CHUNK // LANE):
            r = rvm[b, pl.ds(g * LANE, LANE)]
            loc = r - row_base
            ok = (loc >= 0) & (loc < HALF)
            lvm[b, pl.ds(g * LANE, LANE)] = jnp.where(ok, loc, DUMMY_ROW)

        def _scale(g, carry2):
            vv = vvm[b, pl.ds(g * LANE, LANE)]
            for k in range(LANE):
                v = vv[k]
                e = g * LANE + k
                for d4 in range(D // LANE):
                    sl = pl.ds(d4 * LANE, LANE)
                    gvm[b, e, sl] = gvm[b, e, sl] * v
            return carry2

        lax.fori_loop(0, CHUNK // LANE, _scale, 0)

        @pl.when(t + 2 < NSTEP)
        def _():
            _meta_start(t + 2, b)

        _scatter_start(b)

    def _step2(i2, carry):
        _step(2 * i2, 0)
        _step(2 * i2 + 1, 1)
        return carry

    lax.fori_loop(0, NSTEP // 2, _step2, 0)
    _scatter_wait(0)
    _scatter_wait(1)
    plsc.subcore_barrier()

    out_base = row_base + sid * TILE_ROWS
    pltpu.sync_copy(acc.at[pl.ds(abase, LAST_ROWS)],
                    out.at[pl.ds(out_base, LAST_ROWS)])

    @pl.when(sid < NS - 1)
    def _():
        pltpu.sync_copy(acc.at[pl.ds(abase + LAST_ROWS, TILE_ROWS - LAST_ROWS)],
                        out.at[pl.ds(out_base + LAST_ROWS, TILE_ROWS - LAST_ROWS)])


@functools.cache
def _get_hop_call():
    return pl.kernel(
        _hop_body,
        out_type=jax.ShapeDtypeStruct((N_NODES, D), jnp.float32),
        mesh=plsc.VectorSubcoreMesh(core_axis_name="c", subcore_axis_name="s",
                                    num_cores=NC, num_subcores=NS),
        compiler_params=pltpu.CompilerParams(use_tc_tiling_on_sc=False),
        scratch_types=[
            pltpu.VMEM_SHARED((ACC_ROWS, D), jnp.float32),
            pltpu.VMEM((2, CHUNK), jnp.int32),
            pltpu.VMEM((2, CHUNK), jnp.int32),
            pltpu.VMEM((2, CHUNK), jnp.float32),
            pltpu.VMEM((2, CHUNK), jnp.int32),
            pltpu.VMEM((2, CHUNK, D), jnp.float32),
            pltpu.SemaphoreType.DMA,
            pltpu.SemaphoreType.DMA,
            pltpu.SemaphoreType.DMA,
            pltpu.SemaphoreType.DMA,
            pltpu.SemaphoreType.DMA,
            pltpu.SemaphoreType.DMA,
            pltpu.SemaphoreType.DMA,
        ],
    )

U_PER_W = BATCH // NW
IDX_WORDS = 2 * U_PER_W + NEGS * U_PER_W


def _gather_body(e0, e1, e2, e3, users, pos, negs_t, s_out, p_out, n_out,
                 ivm, gvm, sem):
    core = lax.axis_index("c")
    sid = lax.axis_index("s")
    wid = sid * NC + core
    ub = wid * U_PER_W
    pltpu.sync_copy(users.at[pl.ds(ub, U_PER_W)], ivm.at[pl.ds(0, U_PER_W)])
    pltpu.sync_copy(pos.at[pl.ds(ub, U_PER_W)], ivm.at[pl.ds(U_PER_W, U_PER_W)])
    for j in range(NEGS):
        pltpu.sync_copy(negs_t.at[pl.ds(j * BATCH + ub, U_PER_W)],
                        ivm.at[pl.ds((2 + j) * U_PER_W, U_PER_W)])
    for g in range(U_PER_W // LANE, IDX_WORDS // LANE):
        sl = pl.ds(g * LANE, LANE)
        ivm[sl] = ivm[sl] + N_USERS
    for l, t in enumerate((e0, e1, e2, e3)):
        pltpu.async_copy(t.at[ivm.at[pl.ds(0, U_PER_W)]], gvm, sem).wait()
        pltpu.sync_copy(gvm, s_out.at[l, pl.ds(ub, U_PER_W)])
        pltpu.async_copy(t.at[ivm.at[pl.ds(U_PER_W, U_PER_W)]], gvm, sem).wait()
        pltpu.sync_copy(gvm, p_out.at[l, pl.ds(ub, U_PER_W)])
        for j in range(NEGS):
            pltpu.async_copy(t.at[ivm.at[pl.ds((2 + j) * U_PER_W, U_PER_W)]],
                             gvm, sem).wait()
            pltpu.sync_copy(gvm, n_out.at[l, j, pl.ds(ub, U_PER_W)])


@functools.cache
def _get_gather_call():
    return pl.kernel(
        _gather_body,
        out_type=(
            jax.ShapeDtypeStruct((4, BATCH, D), jnp.float32),
            jax.ShapeDtypeStruct((4, BATCH, D), jnp.float32),
            jax.ShapeDtypeStruct((4, NEGS, BATCH, D), jnp.float32),
        ),
        mesh=plsc.VectorSubcoreMesh(core_axis_name="c", subcore_axis_name="s",
                                    num_cores=NC, num_subcores=NS),
        compiler_params=pltpu.CompilerParams(use_tc_tiling_on_sc=False),
        scratch_types=[
            pltpu.VMEM((IDX_WORDS,), jnp.int32),
            pltpu.VMEM((U_PER_W, D), jnp.float32),
            pltpu.SemaphoreType.DMA,
        ],
    )

BB = 256
GB = BATCH // BB


def _dotT(x, w):
    return lax.dot_general(x, w, (((1,), (1,)), ((), ())),
                           preferred_element_type=jnp.float32)


def _loss_body(factor_ref, s_ref, p_ref, n_ref, wu, bu, wi, bi, wp, bp,
               wn, bn, loss_ref, reg_ref):
    factor = factor_ref[0, 0]
    u_acc = jnp.zeros((BB, D), jnp.float32)
    pos_acc = jnp.zeros((BB, D), jnp.float32)
    neg_acc = jnp.zeros((BB, D), jnp.float32)
    sel0 = jnp.zeros((BB, D), jnp.float32)
    for l in range(4):
        s_l = s_ref[l]
        p_l = p_ref[l]
        gate_p = jax.nn.sigmoid(_dotT(p_l, wi[...]) + bi[...]
                                + _dotT(s_l, wu[...]) + bu[...])
        gated_p = p_l * gate_p
        gp = _dotT(gated_p, wp[...]) + bp[...]
        best = jnp.full((BB, 1), -1e30, jnp.float32)
        bidx = jnp.zeros((BB, 1), jnp.int32)
        for j in range(NEGS):
            n_j = n_ref[l, j]
            gate_n = jax.nn.sigmoid(_dotT(n_j, wn[...]) + bn[...] + gp)
            n_sel = factor * n_j - n_j * gate_n
            sc = jnp.sum(n_sel * s_l, axis=1, keepdims=True)
            upd = sc > best
            bidx = jnp.where(upd, j, bidx)
            best = jnp.where(upd, sc, best)
        sel = jnp.zeros((BB, D), jnp.float32)
        for j in range(NEGS):
            sel = sel + jnp.where(bidx == j, n_ref[l, j], 0.0)
        u_acc = u_acc + s_l
        pos_acc = pos_acc + p_l
        neg_acc = neg_acc + sel
        if l == 0:
            sel0 = sel
            reg_blk = (jnp.sum(s_l * s_l) + jnp.sum(p_l * p_l))
    reg_blk = reg_blk + jnp.sum(sel0 * sel0)
    u_e = u_acc * 0.25
    pos_e = pos_acc * 0.25
    neg_e = neg_acc * 0.25
    d_sc = jnp.sum(u_e * neg_e, axis=1) - jnp.sum(u_e * pos_e, axis=1)
    blk_loss = jnp.sum(jnp.log(1.0 + jnp.exp(d_sc)))

    @pl.when(pl.program_id(0) == 0)
    def _():
        loss_ref[0, 0] = 0.0
        reg_ref[0, 0] = 0.0

    loss_ref[0, 0] += blk_loss
    reg_ref[0, 0] += reg_blk


_loss_call = pl.pallas_call(
    _loss_body,
    grid=(GB,),
    in_specs=[
        pl.BlockSpec(memory_space=pltpu.SMEM),
        pl.BlockSpec((4, BB, D), lambda i: (0, i, 0)),
        pl.BlockSpec((4, BB, D), lambda i: (0, i, 0)),
        pl.BlockSpec((4, NEGS, BB, D), lambda i: (0, 0, i, 0)),
        pl.BlockSpec((D, D), lambda i: (0, 0)),
        pl.BlockSpec((1, D), lambda i: (0, 0)),
        pl.BlockSpec((D, D), lambda i: (0, 0)),
        pl.BlockSpec((1, D), lambda i: (0, 0)),
        pl.BlockSpec((D, D), lambda i: (0, 0)),
        pl.BlockSpec((1, D), lambda i: (0, 0)),
        pl.BlockSpec((D, D), lambda i: (0, 0)),
        pl.BlockSpec((1, D), lambda i: (0, 0)),
    ],
    out_specs=[
        pl.BlockSpec(memory_space=pltpu.SMEM),
        pl.BlockSpec(memory_space=pltpu.SMEM),
    ],
    out_shape=[
        jax.ShapeDtypeStruct((1, 1), jnp.float32),
        jax.ShapeDtypeStruct((1, 1), jnp.float32),
    ],
)


def kernel(cur_epoch, users, pos_items, neg_items, adj_rows, adj_cols,
           adj_vals, user_embed, item_embed,
           W_user_gate, b_user_gate, W_item_gate, b_item_gate,
           W_pos_gate, b_pos_gate, W_neg_gate, b_neg_gate):
    pad = NNZ_PAD - NNZ
    rows_p = jnp.concatenate([adj_rows, jnp.zeros((pad,), jnp.int32)])
    cols_p = jnp.concatenate([adj_cols, jnp.zeros((pad,), jnp.int32)])
    vals_p = jnp.concatenate([adj_vals, jnp.zeros((pad,), jnp.float32)])
    e0 = jnp.concatenate([user_embed, item_embed], axis=0)
    hop = _get_hop_call()
    e1 = hop(e0, rows_p, cols_p, vals_p)
    e2 = hop(e1, rows_p, cols_p, vals_p)
    e3 = hop(e2, rows_p, cols_p, vals_p)
    negs_t = neg_items.T.reshape(-1)
    s_all, p_all, n_all = _get_gather_call()(e0, e1, e2, e3, users,
                                             pos_items, negs_t)
    factor = (1.0 - jnp.minimum(
        1.0, jnp.asarray(cur_epoch).astype(jnp.float32) / WARMUP)).reshape(1, 1)
    loss_sum, reg_sum = _loss_call(
        factor, s_all, p_all, n_all,
        W_user_gate, b_user_gate.reshape(1, D),
        W_item_gate, b_item_gate.reshape(1, D),
        W_pos_gate, b_pos_gate.reshape(1, D),
        W_neg_gate, b_neg_gate.reshape(1, D))
    mf_loss = loss_sum[0, 0] / BATCH
    emb_loss = (DECAY / (2.0 * BATCH)) * reg_sum[0, 0]
    return mf_loss + emb_loss, mf_loss, emb_loss


# speedup vs baseline: 2.9757x; 1.2917x over previous
import functools

import jax
import jax.numpy as jnp
from jax import lax
from jax.experimental import pallas as pl
from jax.experimental.pallas import tpu as pltpu
from jax.experimental.pallas import tpu_sc as plsc

N_USERS = 10000
N_ITEMS = 40000
N_NODES = 50000
D = 64
NNZ = 800000
BATCH = 4096
NEGS = 16
WARMUP = 100.0
DECAY = 1e-4

NC = 2
NS = 16
NW = NC * NS
LANE = 16

HALF = N_NODES // NC
TILE_ROWS = 1568
ACC_ROWS = NS * TILE_ROWS
DUMMY_ROW = ACC_ROWS - 8
LAST_ROWS = HALF - (NS - 1) * TILE_ROWS
EPT = 50688
NNZ_PAD = NS * EPT
CHUNK = 128
NSTEP = EPT // CHUNK


def _hop_body(table, rows, cols, vals, out, acc,
              cvm, rvm, vvm, lvm, gvm, gsem0, gsem1, gsem2, msem0, msem1,
              msem2, ssem0, ssem1, ssem2):
    core = lax.axis_index("c")
    sid = lax.axis_index("s")
    row_base = core * HALF
    gsem = (gsem0, gsem1, gsem2)
    msem = (msem0, msem1, msem2)
    ssem = (ssem0, ssem1, ssem2)

    def _zero_row(i, carry):
        for d4 in range(D // LANE):
            gvm[0, i, pl.ds(d4 * LANE, LANE)] = jnp.zeros((LANE,), jnp.float32)
        return carry

    lax.fori_loop(0, CHUNK, _zero_row, 0)
    abase = sid * TILE_ROWS
    for t in range(TILE_ROWS // CHUNK):
        pltpu.sync_copy(gvm.at[0], acc.at[pl.ds(abase + t * CHUNK, CHUNK)])
    _zrem = TILE_ROWS % CHUNK
    if _zrem:
        pltpu.sync_copy(gvm.at[0, pl.ds(0, _zrem)],
                        acc.at[pl.ds(abase + TILE_ROWS - _zrem, _zrem)])
    plsc.subcore_barrier()

    ebase0 = sid * EPT

    def _meta_start(t, b):
        off = ebase0 + t * CHUNK
        pltpu.async_copy(cols.at[pl.ds(off, CHUNK)], cvm.at[b], msem[b])
        pltpu.async_copy(rows.at[pl.ds(off, CHUNK)], rvm.at[b], msem[b])
        pltpu.async_copy(vals.at[pl.ds(off, CHUNK)], vvm.at[b], msem[b])

    def _meta_wait(b):
        pltpu.make_async_copy(cols.at[pl.ds(0, CHUNK)], cvm.at[b], msem[b]).wait()
        pltpu.make_async_copy(rows.at[pl.ds(0, CHUNK)], rvm.at[b], msem[b]).wait()
        pltpu.make_async_copy(vals.at[pl.ds(0, CHUNK)], vvm.at[b], msem[b]).wait()

    def _gather_start(b):
        pltpu.async_copy(table.at[cvm.at[b]], gvm.at[b], gsem[b])

    def _gather_wait(b):
        pltpu.make_async_copy(table.at[cvm.at[b]], gvm.at[b], gsem[b]).wait()

    def _scatter_start(b):
        pltpu.async_copy(gvm.at[b], acc.at[lvm.at[b]], ssem[b], add=True)

    def _scatter_wait(b):
        pltpu.make_async_copy(gvm.at[b], acc.at[lvm.at[b]], ssem[b]).wait()

    _meta_start(0, 0)
    _meta_start(1, 1)
    _meta_start(2, 2)
    _meta_wait(0)
    _gather_start(0)
    _meta_wait(1)
    _gather_start(1)

    def _step(t, b):
        _gather_wait(b)

        nb = (b + 2) % 3

        @pl.when((t >= 1) & (t + 2 < NSTEP))
        def _():
            _scatter_wait(nb)

        @pl.when(t + 2 < NSTEP)
        def _():
            _meta_wait(nb)
            _gather_start(nb)

        for g in range(CHUNK // LANE):
            r = rvm[b, pl.ds(g * LANE, LANE)]
            loc = r - row_base
            ok = (loc >= 0) & (loc < HALF)
            lvm[b, pl.ds(g * LANE, LANE)] = jnp.where(ok, loc, DUMMY_ROW)

        def _scale(g, carry2):
            vv = vvm[b, pl.ds(g * LANE, LANE)]
            for k in range(LANE):
                v = vv[k]
                e = g * LANE + k
                for d4 in range(D // LANE):
                    sl = pl.ds(d4 * LANE, LANE)
                    gvm[b, e, sl] = gvm[b, e, sl] * v
            return carry2

        lax.fori_loop(0, CHUNK // LANE, _scale, 0)

        @pl.when(t + 3 < NSTEP)
        def _():
            _meta_start(t + 3, b)

        _scatter_start(b)

    def _step3(i3, carry):
        _step(3 * i3, 0)
        _step(3 * i3 + 1, 1)
        _step(3 * i3 + 2, 2)
        return carry

    lax.fori_loop(0, NSTEP // 3, _step3, 0)
    _scatter_wait(0)
    _scatter_wait(1)
    _scatter_wait(2)
    plsc.subcore_barrier()

    out_base = row_base + sid * TILE_ROWS
    pltpu.sync_copy(acc.at[pl.ds(abase, LAST_ROWS)],
                    out.at[pl.ds(out_base, LAST_ROWS)])

    @pl.when(sid < NS - 1)
    def _():
        pltpu.sync_copy(acc.at[pl.ds(abase + LAST_ROWS, TILE_ROWS - LAST_ROWS)],
                        out.at[pl.ds(out_base + LAST_ROWS, TILE_ROWS - LAST_ROWS)])


@functools.cache
def _get_hop_call():
    return pl.kernel(
        _hop_body,
        out_type=jax.ShapeDtypeStruct((N_NODES, D), jnp.float32),
        mesh=plsc.VectorSubcoreMesh(core_axis_name="c", subcore_axis_name="s",
                                    num_cores=NC, num_subcores=NS),
        compiler_params=pltpu.CompilerParams(use_tc_tiling_on_sc=False),
        scratch_types=[
            pltpu.VMEM_SHARED((ACC_ROWS, D), jnp.float32),
            pltpu.VMEM((3, CHUNK), jnp.int32),
            pltpu.VMEM((3, CHUNK), jnp.int32),
            pltpu.VMEM((3, CHUNK), jnp.float32),
            pltpu.VMEM((3, CHUNK), jnp.int32),
            pltpu.VMEM((3, CHUNK, D), jnp.float32),
            pltpu.SemaphoreType.DMA,
            pltpu.SemaphoreType.DMA,
            pltpu.SemaphoreType.DMA,
            pltpu.SemaphoreType.DMA,
            pltpu.SemaphoreType.DMA,
            pltpu.SemaphoreType.DMA,
            pltpu.SemaphoreType.DMA,
            pltpu.SemaphoreType.DMA,
            pltpu.SemaphoreType.DMA,
        ],
    )

U_PER_W = BATCH // NW
IDX_WORDS = 2 * U_PER_W + NEGS * U_PER_W


def _gather_body(e0, e1, e2, e3, users, pos, negs_t, s_out, p_out, n_out,
                 ivm, gvm, sem):
    core = lax.axis_index("c")
    sid = lax.axis_index("s")
    wid = sid * NC + core
    ub = wid * U_PER_W
    pltpu.sync_copy(users.at[pl.ds(ub, U_PER_W)], ivm.at[pl.ds(0, U_PER_W)])
    pltpu.sync_copy(pos.at[pl.ds(ub, U_PER_W)], ivm.at[pl.ds(U_PER_W, U_PER_W)])
    for j in range(NEGS):
        pltpu.sync_copy(negs_t.at[pl.ds(j * BATCH + ub, U_PER_W)],
                        ivm.at[pl.ds((2 + j) * U_PER_W, U_PER_W)])
    for g in range(U_PER_W // LANE, IDX_WORDS // LANE):
        sl = pl.ds(g * LANE, LANE)
        ivm[sl] = ivm[sl] + N_USERS
    for l, t in enumerate((e0, e1, e2, e3)):
        pltpu.async_copy(t.at[ivm.at[pl.ds(0, U_PER_W)]], gvm, sem).wait()
        pltpu.sync_copy(gvm, s_out.at[l, pl.ds(ub, U_PER_W)])
        pltpu.async_copy(t.at[ivm.at[pl.ds(U_PER_W, U_PER_W)]], gvm, sem).wait()
        pltpu.sync_copy(gvm, p_out.at[l, pl.ds(ub, U_PER_W)])
        for j in range(NEGS):
            pltpu.async_copy(t.at[ivm.at[pl.ds((2 + j) * U_PER_W, U_PER_W)]],
                             gvm, sem).wait()
            pltpu.sync_copy(gvm, n_out.at[l, j, pl.ds(ub, U_PER_W)])


@functools.cache
def _get_gather_call():
    return pl.kernel(
        _gather_body,
        out_type=(
            jax.ShapeDtypeStruct((4, BATCH, D), jnp.float32),
            jax.ShapeDtypeStruct((4, BATCH, D), jnp.float32),
            jax.ShapeDtypeStruct((4, NEGS, BATCH, D), jnp.float32),
        ),
        mesh=plsc.VectorSubcoreMesh(core_axis_name="c", subcore_axis_name="s",
                                    num_cores=NC, num_subcores=NS),
        compiler_params=pltpu.CompilerParams(use_tc_tiling_on_sc=False),
        scratch_types=[
            pltpu.VMEM((IDX_WORDS,), jnp.int32),
            pltpu.VMEM((U_PER_W, D), jnp.float32),
            pltpu.SemaphoreType.DMA,
        ],
    )

BB = 256
GB = BATCH // BB


def _dotT(x, w):
    return lax.dot_general(x, w, (((1,), (1,)), ((), ())),
                           preferred_element_type=jnp.float32)


def _loss_body(factor_ref, s_ref, p_ref, n_ref, wu, bu, wi, bi, wp, bp,
               wn, bn, loss_ref, reg_ref):
    factor = factor_ref[0, 0]
    u_acc = jnp.zeros((BB, D), jnp.float32)
    pos_acc = jnp.zeros((BB, D), jnp.float32)
    neg_acc = jnp.zeros((BB, D), jnp.float32)
    sel0 = jnp.zeros((BB, D), jnp.float32)
    for l in range(4):
        s_l = s_ref[l]
        p_l = p_ref[l]
        gate_p = jax.nn.sigmoid(_dotT(p_l, wi[...]) + bi[...]
                                + _dotT(s_l, wu[...]) + bu[...])
        gated_p = p_l * gate_p
        gp = _dotT(gated_p, wp[...]) + bp[...]
        best = jnp.full((BB, 1), -1e30, jnp.float32)
        bidx = jnp.zeros((BB, 1), jnp.int32)
        for j in range(NEGS):
            n_j = n_ref[l, j]
            gate_n = jax.nn.sigmoid(_dotT(n_j, wn[...]) + bn[...] + gp)
            n_sel = factor * n_j - n_j * gate_n
            sc = jnp.sum(n_sel * s_l, axis=1, keepdims=True)
            upd = sc > best
            bidx = jnp.where(upd, j, bidx)
            best = jnp.where(upd, sc, best)
        sel = jnp.zeros((BB, D), jnp.float32)
        for j in range(NEGS):
            sel = sel + jnp.where(bidx == j, n_ref[l, j], 0.0)
        u_acc = u_acc + s_l
        pos_acc = pos_acc + p_l
        neg_acc = neg_acc + sel
        if l == 0:
            sel0 = sel
            reg_blk = (jnp.sum(s_l * s_l) + jnp.sum(p_l * p_l))
    reg_blk = reg_blk + jnp.sum(sel0 * sel0)
    u_e = u_acc * 0.25
    pos_e = pos_acc * 0.25
    neg_e = neg_acc * 0.25
    d_sc = jnp.sum(u_e * neg_e, axis=1) - jnp.sum(u_e * pos_e, axis=1)
    blk_loss = jnp.sum(jnp.log(1.0 + jnp.exp(d_sc)))

    @pl.when(pl.program_id(0) == 0)
    def _():
        loss_ref[0, 0] = 0.0
        reg_ref[0, 0] = 0.0

    loss_ref[0, 0] += blk_loss
    reg_ref[0, 0] += reg_blk


_loss_call = pl.pallas_call(
    _loss_body,
    grid=(GB,),
    in_specs=[
        pl.BlockSpec(memory_space=pltpu.SMEM),
        pl.BlockSpec((4, BB, D), lambda i: (0, i, 0)),
        pl.BlockSpec((4, BB, D), lambda i: (0, i, 0)),
        pl.BlockSpec((4, NEGS, BB, D), lambda i: (0, 0, i, 0)),
        pl.BlockSpec((D, D), lambda i: (0, 0)),
        pl.BlockSpec((1, D), lambda i: (0, 0)),
        pl.BlockSpec((D, D), lambda i: (0, 0)),
        pl.BlockSpec((1, D), lambda i: (0, 0)),
        pl.BlockSpec((D, D), lambda i: (0, 0)),
        pl.BlockSpec((1, D), lambda i: (0, 0)),
        pl.BlockSpec((D, D), lambda i: (0, 0)),
        pl.BlockSpec((1, D), lambda i: (0, 0)),
    ],
    out_specs=[
        pl.BlockSpec(memory_space=pltpu.SMEM),
        pl.BlockSpec(memory_space=pltpu.SMEM),
    ],
    out_shape=[
        jax.ShapeDtypeStruct((1, 1), jnp.float32),
        jax.ShapeDtypeStruct((1, 1), jnp.float32),
    ],
)


def kernel(cur_epoch, users, pos_items, neg_items, adj_rows, adj_cols,
           adj_vals, user_embed, item_embed,
           W_user_gate, b_user_gate, W_item_gate, b_item_gate,
           W_pos_gate, b_pos_gate, W_neg_gate, b_neg_gate):
    pad = NNZ_PAD - NNZ
    rows_p = jnp.concatenate([adj_rows, jnp.zeros((pad,), jnp.int32)])
    cols_p = jnp.concatenate([adj_cols, jnp.zeros((pad,), jnp.int32)])
    vals_p = jnp.concatenate([adj_vals, jnp.zeros((pad,), jnp.float32)])
    e0 = jnp.concatenate([user_embed, item_embed], axis=0)
    hop = _get_hop_call()
    e1 = hop(e0, rows_p, cols_p, vals_p)
    e2 = hop(e1, rows_p, cols_p, vals_p)
    e3 = hop(e2, rows_p, cols_p, vals_p)
    negs_t = neg_items.T.reshape(-1)
    s_all, p_all, n_all = _get_gather_call()(e0, e1, e2, e3, users,
                                             pos_items, negs_t)
    factor = (1.0 - jnp.minimum(
        1.0, jnp.asarray(cur_epoch).astype(jnp.float32) / WARMUP)).reshape(1, 1)
    loss_sum, reg_sum = _loss_call(
        factor, s_all, p_all, n_all,
        W_user_gate, b_user_gate.reshape(1, D),
        W_item_gate, b_item_gate.reshape(1, D),
        W_pos_gate, b_pos_gate.reshape(1, D),
        W_neg_gate, b_neg_gate.reshape(1, D))
    mf_loss = loss_sum[0, 0] / BATCH
    emb_loss = (DECAY / (2.0 * BATCH)) * reg_sum[0, 0]
    return mf_loss + emb_loss, mf_loss, emb_loss


# speedup vs baseline: 3.0240x; 1.0162x over previous
import functools

import jax
import jax.numpy as jnp
from jax import lax
from jax.experimental import pallas as pl
from jax.experimental.pallas import tpu as pltpu
from jax.experimental.pallas import tpu_sc as plsc

N_USERS = 10000
N_ITEMS = 40000
N_NODES = 50000
D = 64
NNZ = 800000
BATCH = 4096
NEGS = 16
WARMUP = 100.0
DECAY = 1e-4

NC = 2
NS = 16
NW = NC * NS
LANE = 16

HALF = N_NODES // NC
TILE_ROWS = 1568
ACC_ROWS = NS * TILE_ROWS
DUMMY_ROW = ACC_ROWS - 8
LAST_ROWS = HALF - (NS - 1) * TILE_ROWS
EPT = 50688
NNZ_PAD = NS * EPT
CHUNK = 128
NSTEP = EPT // CHUNK


SUPER = 3
G_TILE = NSTEP // SUPER
NMETA = NNZ_PAD // (SUPER * CHUNK)


def _hop_body(table, meta, out, acc, mvm, lvm, gvm,
              gsem0, gsem1, gsem2, msem0, msem1, ssem0, ssem1, ssem2):
    core = lax.axis_index("c")
    sid = lax.axis_index("s")
    row_base = core * HALF
    gsem = (gsem0, gsem1, gsem2)
    msem = (msem0, msem1)
    ssem = (ssem0, ssem1, ssem2)

    def _zero_row(i, carry):
        for d4 in range(D // LANE):
            gvm[0, i, pl.ds(d4 * LANE, LANE)] = jnp.zeros((LANE,), jnp.float32)
        return carry

    lax.fori_loop(0, CHUNK, _zero_row, 0)
    abase = sid * TILE_ROWS
    for t in range(TILE_ROWS // CHUNK):
        pltpu.sync_copy(gvm.at[0], acc.at[pl.ds(abase + t * CHUNK, CHUNK)])
    _zrem = TILE_ROWS % CHUNK
    if _zrem:
        pltpu.sync_copy(gvm.at[0, pl.ds(0, _zrem)],
                        acc.at[pl.ds(abase + TILE_ROWS - _zrem, _zrem)])
    plsc.subcore_barrier()

    gbase0 = sid * G_TILE

    def _meta_start(g, mb):
        pltpu.async_copy(meta.at[gbase0 + g], mvm.at[mb], msem[mb])

    def _meta_wait(mb):
        pltpu.make_async_copy(meta.at[gbase0], mvm.at[mb], msem[mb]).wait()

    def _gather_start(gb, mb, j):
        pltpu.async_copy(table.at[mvm.at[mb, 0, j]], gvm.at[gb], gsem[gb])

    def _gather_wait(gb, mb, j):
        pltpu.make_async_copy(table.at[mvm.at[mb, 0, j]], gvm.at[gb],
                              gsem[gb]).wait()

    def _scatter_start(gb):
        pltpu.async_copy(gvm.at[gb], acc.at[lvm.at[gb]], ssem[gb], add=True)

    def _scatter_wait(gb):
        pltpu.make_async_copy(gvm.at[gb], acc.at[lvm.at[gb]], ssem[gb]).wait()

    _meta_start(0, 0)
    _meta_start(1, 1)
    _meta_wait(0)
    _gather_start(0, 0, 0)
    _gather_start(1, 0, 1)

    def _step(i12, k):
        t = 12 * i12 + k
        gb = k % 3
        mb = (k // SUPER) % 2
        j = k % SUPER
        k2 = (k + 2) % 12
        gb2 = k2 % 3
        mb2 = (k2 // SUPER) % 2
        j2 = k2 % SUPER

        _gather_wait(gb, mb, j)

        @pl.when((t >= 1) & (t + 2 < NSTEP))
        def _():
            _scatter_wait(gb2)

        @pl.when(t + 2 < NSTEP)
        def _():
            if j2 == 0:
                _meta_wait(mb2)
            _gather_start(gb2, mb2, j2)

        for g in range(CHUNK // LANE):
            r = mvm[mb, 1, j, pl.ds(g * LANE, LANE)]
            loc = r - row_base
            ok = (loc >= 0) & (loc < HALF)
            lvm[gb, pl.ds(g * LANE, LANE)] = jnp.where(ok, loc, DUMMY_ROW)

        def _scale(g, carry2):
            vv = plsc.bitcast(mvm[mb, 2, j, pl.ds(g * LANE, LANE)],
                              jnp.float32)
            for k_ in range(LANE):
                v = vv[k_]
                e = g * LANE + k_
                for d4 in range(D // LANE):
                    sl = pl.ds(d4 * LANE, LANE)
                    gvm[gb, e, sl] = gvm[gb, e, sl] * v
            return carry2

        lax.fori_loop(0, CHUNK // LANE, _scale, 0)

        if k % SUPER == SUPER - 1:
            g_next = (12 // SUPER) * i12 + (k // SUPER) + 2

            @pl.when(g_next < G_TILE)
            def _():
                _meta_start(g_next, mb)

        _scatter_start(gb)

    def _step12(i12, carry):
        for k in range(12):
            _step(i12, k)
        return carry

    lax.fori_loop(0, NSTEP // 12, _step12, 0)
    _scatter_wait(0)
    _scatter_wait(1)
    _scatter_wait(2)
    plsc.subcore_barrier()

    out_base = row_base + sid * TILE_ROWS
    pltpu.sync_copy(acc.at[pl.ds(abase, LAST_ROWS)],
                    out.at[pl.ds(out_base, LAST_ROWS)])

    @pl.when(sid < NS - 1)
    def _():
        pltpu.sync_copy(acc.at[pl.ds(abase + LAST_ROWS, TILE_ROWS - LAST_ROWS)],
                        out.at[pl.ds(out_base + LAST_ROWS, TILE_ROWS - LAST_ROWS)])


@functools.cache
def _get_hop_call():
    return pl.kernel(
        _hop_body,
        out_type=jax.ShapeDtypeStruct((N_NODES, D), jnp.float32),
        mesh=plsc.VectorSubcoreMesh(core_axis_name="c", subcore_axis_name="s",
                                    num_cores=NC, num_subcores=NS),
        compiler_params=pltpu.CompilerParams(use_tc_tiling_on_sc=False,
                                             needs_layout_passes=False),
        scratch_types=[
            pltpu.VMEM_SHARED((ACC_ROWS, D), jnp.float32),
            pltpu.VMEM((2, 3, SUPER, CHUNK), jnp.int32),
            pltpu.VMEM((3, CHUNK), jnp.int32),
            pltpu.VMEM((3, CHUNK, D), jnp.float32),
        ] + [pltpu.SemaphoreType.DMA] * 8,
    )

U_PER_W = BATCH // NW
IDX_WORDS = 2 * U_PER_W + NEGS * U_PER_W


def _gather_body(e0, e1, e2, e3, users, pos, negs_t, s_out, p_out, n_out,
                 ivm, gvm, sem):
    core = lax.axis_index("c")
    sid = lax.axis_index("s")
    wid = sid * NC + core
    ub = wid * U_PER_W
    pltpu.sync_copy(users.at[pl.ds(ub, U_PER_W)], ivm.at[pl.ds(0, U_PER_W)])
    pltpu.sync_copy(pos.at[pl.ds(ub, U_PER_W)], ivm.at[pl.ds(U_PER_W, U_PER_W)])
    for j in range(NEGS):
        pltpu.sync_copy(negs_t.at[pl.ds(j * BATCH + ub, U_PER_W)],
                        ivm.at[pl.ds((2 + j) * U_PER_W, U_PER_W)])
    for g in range(U_PER_W // LANE, IDX_WORDS // LANE):
        sl = pl.ds(g * LANE, LANE)
        ivm[sl] = ivm[sl] + N_USERS
    for l, t in enumerate((e0, e1, e2, e3)):
        pltpu.async_copy(t.at[ivm.at[pl.ds(0, U_PER_W)]], gvm, sem).wait()
        pltpu.sync_copy(gvm, s_out.at[l, pl.ds(ub, U_PER_W)])
        pltpu.async_copy(t.at[ivm.at[pl.ds(U_PER_W, U_PER_W)]], gvm, sem).wait()
        pltpu.sync_copy(gvm, p_out.at[l, pl.ds(ub, U_PER_W)])
        for j in range(NEGS):
            pltpu.async_copy(t.at[ivm.at[pl.ds((2 + j) * U_PER_W, U_PER_W)]],
                             gvm, sem).wait()
            pltpu.sync_copy(gvm, n_out.at[l, j, pl.ds(ub, U_PER_W)])


@functools.cache
def _get_gather_call():
    return pl.kernel(
        _gather_body,
        out_type=(
            jax.ShapeDtypeStruct((4, BATCH, D), jnp.float32),
            jax.ShapeDtypeStruct((4, BATCH, D), jnp.float32),
            jax.ShapeDtypeStruct((4, NEGS, BATCH, D), jnp.float32),
        ),
        mesh=plsc.VectorSubcoreMesh(core_axis_name="c", subcore_axis_name="s",
                                    num_cores=NC, num_subcores=NS),
        compiler_params=pltpu.CompilerParams(use_tc_tiling_on_sc=False),
        scratch_types=[
            pltpu.VMEM((IDX_WORDS,), jnp.int32),
            pltpu.VMEM((U_PER_W, D), jnp.float32),
            pltpu.SemaphoreType.DMA,
        ],
    )

BB = 256
GB = BATCH // BB


def _dotT(x, w):
    return lax.dot_general(x, w, (((1,), (1,)), ((), ())),
                           preferred_element_type=jnp.float32)


def _loss_body(factor_ref, s_ref, p_ref, n_ref, wu, bu, wi, bi, wp, bp,
               wn, bn, loss_ref, reg_ref):
    factor = factor_ref[0, 0]
    u_acc = jnp.zeros((BB, D), jnp.float32)
    pos_acc = jnp.zeros((BB, D), jnp.float32)
    neg_acc = jnp.zeros((BB, D), jnp.float32)
    sel0 = jnp.zeros((BB, D), jnp.float32)
    for l in range(4):
        s_l = s_ref[l]
        p_l = p_ref[l]
        gate_p = jax.nn.sigmoid(_dotT(p_l, wi[...]) + bi[...]
                                + _dotT(s_l, wu[...]) + bu[...])
        gated_p = p_l * gate_p
        gp = _dotT(gated_p, wp[...]) + bp[...]
        best = jnp.full((BB, 1), -1e30, jnp.float32)
        bidx = jnp.zeros((BB, 1), jnp.int32)
        for j in range(NEGS):
            n_j = n_ref[l, j]
            gate_n = jax.nn.sigmoid(_dotT(n_j, wn[...]) + bn[...] + gp)
            n_sel = factor * n_j - n_j * gate_n
            sc = jnp.sum(n_sel * s_l, axis=1, keepdims=True)
            upd = sc > best
            bidx = jnp.where(upd, j, bidx)
            best = jnp.where(upd, sc, best)
        sel = jnp.zeros((BB, D), jnp.float32)
        for j in range(NEGS):
            sel = sel + jnp.where(bidx == j, n_ref[l, j], 0.0)
        u_acc = u_acc + s_l
        pos_acc = pos_acc + p_l
        neg_acc = neg_acc + sel
        if l == 0:
            sel0 = sel
            reg_blk = (jnp.sum(s_l * s_l) + jnp.sum(p_l * p_l))
    reg_blk = reg_blk + jnp.sum(sel0 * sel0)
    u_e = u_acc * 0.25
    pos_e = pos_acc * 0.25
    neg_e = neg_acc * 0.25
    d_sc = jnp.sum(u_e * neg_e, axis=1) - jnp.sum(u_e * pos_e, axis=1)
    blk_loss = jnp.sum(jnp.log(1.0 + jnp.exp(d_sc)))

    @pl.when(pl.program_id(0) == 0)
    def _():
        loss_ref[0, 0] = 0.0
        reg_ref[0, 0] = 0.0

    loss_ref[0, 0] += blk_loss
    reg_ref[0, 0] += reg_blk


_loss_call = pl.pallas_call(
    _loss_body,
    grid=(GB,),
    in_specs=[
        pl.BlockSpec(memory_space=pltpu.SMEM),
        pl.BlockSpec((4, BB, D), lambda i: (0, i, 0)),
        pl.BlockSpec((4, BB, D), lambda i: (0, i, 0)),
        pl.BlockSpec((4, NEGS, BB, D), lambda i: (0, 0, i, 0)),
        pl.BlockSpec((D, D), lambda i: (0, 0)),
        pl.BlockSpec((1, D), lambda i: (0, 0)),
        pl.BlockSpec((D, D), lambda i: (0, 0)),
        pl.BlockSpec((1, D), lambda i: (0, 0)),
        pl.BlockSpec((D, D), lambda i: (0, 0)),
        pl.BlockSpec((1, D), lambda i: (0, 0)),
        pl.BlockSpec((D, D), lambda i: (0, 0)),
        pl.BlockSpec((1, D), lambda i: (0, 0)),
    ],
    out_specs=[
        pl.BlockSpec(memory_space=pltpu.SMEM),
        pl.BlockSpec(memory_space=pltpu.SMEM),
    ],
    out_shape=[
        jax.ShapeDtypeStruct((1, 1), jnp.float32),
        jax.ShapeDtypeStruct((1, 1), jnp.float32),
    ],
)


def kernel(cur_epoch, users, pos_items, neg_items, adj_rows, adj_cols,
           adj_vals, user_embed, item_embed,
           W_user_gate, b_user_gate, W_item_gate, b_item_gate,
           W_pos_gate, b_pos_gate, W_neg_gate, b_neg_gate):
    pad = NNZ_PAD - NNZ
    rows_p = jnp.concatenate([adj_rows, jnp.zeros((pad,), jnp.int32)])
    cols_p = jnp.concatenate([adj_cols, jnp.zeros((pad,), jnp.int32)])
    vals_p = jnp.concatenate([adj_vals, jnp.zeros((pad,), jnp.float32)])
    meta = jnp.stack([
        cols_p.reshape(NMETA, SUPER, CHUNK),
        rows_p.reshape(NMETA, SUPER, CHUNK),
        lax.bitcast_convert_type(vals_p, jnp.int32).reshape(NMETA, SUPER,
                                                            CHUNK),
    ], axis=1)
    e0 = jnp.concatenate([user_embed, item_embed], axis=0)
    hop = _get_hop_call()
    e1 = hop(e0, meta)
    e2 = hop(e1, meta)
    e3 = hop(e2, meta)
    negs_t = neg_items.T.reshape(-1)
    s_all, p_all, n_all = _get_gather_call()(e0, e1, e2, e3, users,
                                             pos_items, negs_t)
    factor = (1.0 - jnp.minimum(
        1.0, jnp.asarray(cur_epoch).astype(jnp.float32) / WARMUP)).reshape(1, 1)
    loss_sum, reg_sum = _loss_call(
        factor, s_all, p_all, n_all,
        W_user_gate, b_user_gate.reshape(1, D),
        W_item_gate, b_item_gate.reshape(1, D),
        W_pos_gate, b_pos_gate.reshape(1, D),
        W_neg_gate, b_neg_gate.reshape(1, D))
    mf_loss = loss_sum[0, 0] / BATCH
    emb_loss = (DECAY / (2.0 * BATCH)) * reg_sum[0, 0]
    return mf_loss + emb_loss, mf_loss, emb_loss


# speedup vs baseline: 3.1326x; 1.0359x over previous
import functools

import jax
import jax.numpy as jnp
from jax import lax
from jax.experimental import pallas as pl
from jax.experimental.pallas import tpu as pltpu
from jax.experimental.pallas import tpu_sc as plsc

N_USERS = 10000
N_ITEMS = 40000
N_NODES = 50000
D = 64
NNZ = 800000
BATCH = 4096
NEGS = 16
WARMUP = 100.0
DECAY = 1e-4

NC = 2
NS = 16
NW = NC * NS
LANE = 16

HALF = N_NODES // NC
TILE_ROWS = 1568
ACC_ROWS = NS * TILE_ROWS
DUMMY_ROW = ACC_ROWS - 8
LAST_ROWS = HALF - (NS - 1) * TILE_ROWS
EPT = 50688
NNZ_PAD = NS * EPT
CHUNK = 128
NSTEP = EPT // CHUNK


SUPER = 3
EPW = NNZ_PAD // NW
PCHUNK = EPW // 3
CBUF_E = PCHUNK + 16
RBLK = 70
RCAP = RBLK * SUPER * CHUNK
NBLK_ALL = NW * RBLK


def _prep_body(cols, rows, vals, pcols, prows, pvals, counts,
               civ, riv, viv, c0b, r0b, v0b, c1b, r1b, v1b, zb, cb):
    core = lax.axis_index("c")
    sid = lax.axis_index("s")
    wid = sid * NC + core
    zero = jnp.zeros((LANE,), jnp.int32)

    def _z(i, carry):
        zb[pl.ds(i * LANE, LANE)] = zero
        return carry

    lax.fori_loop(0, 768 // LANE, _z, 0)

    offs = [0, 0]
    bufs = ((c0b, r0b, v0b), (c1b, r1b, v1b))
    outs = (pcols, prows, pvals)
    for c in range(3):
        ibase = wid * EPW + c * PCHUNK
        pltpu.sync_copy(cols.at[pl.ds(ibase, PCHUNK)], civ)
        pltpu.sync_copy(rows.at[pl.ds(ibase, PCHUNK)], riv)
        pltpu.sync_copy(vals.at[pl.ds(ibase, PCHUNK)], viv)

        def _part(g, carry):
            p0, p1 = carry
            sl = pl.ds(g * LANE, LANE)
            cc = civ[sl]
            rr = riv[sl]
            vv = viv[sl]
            m0 = rr < HALF
            m1 = jnp.logical_not(m0)
            plsc.store_compressed(c0b.at[pl.ds(p0, LANE)], cc, mask=m0)
            plsc.store_compressed(r0b.at[pl.ds(p0, LANE)], rr, mask=m0)
            plsc.store_compressed(v0b.at[pl.ds(p0, LANE)], vv, mask=m0)
            plsc.store_compressed(c1b.at[pl.ds(p1, LANE)], cc, mask=m1)
            plsc.store_compressed(r1b.at[pl.ds(p1, LANE)], rr - HALF, mask=m1)
            plsc.store_compressed(v1b.at[pl.ds(p1, LANE)], vv, mask=m1)
            pc = plsc.all_reduce_population_count(m0)[0]
            return (p0 + pc, p1 + (LANE - pc))

        p0, p1 = lax.fori_loop(0, PCHUNK // LANE, _part, (0, 0))
        for h, (cB, rB, vB) in enumerate(bufs):
            p = (p0, p1)[h]
            cB[pl.ds(p, LANE)] = zero
            rB[pl.ds(p, LANE)] = zero
            vB[pl.ds(p, LANE)] = zero
            pr = ((p + 7) // 8) * 8
            obase = wid * RCAP + offs[h]
            pltpu.sync_copy(cB, pcols.at[h, pl.ds(obase, CBUF_E)])
            pltpu.sync_copy(rB, prows.at[h, pl.ds(obase, CBUF_E)])
            pltpu.sync_copy(vB, pvals.at[h, pl.ds(obase, CBUF_E)])
            offs[h] = offs[h] + pr

    for h in range(2):
        obase = wid * RCAP + offs[h]
        for o in outs:
            pltpu.sync_copy(zb, o.at[h, pl.ds(obase, 768)])
        cnt = ((offs[h] + 767) // 768) * 768
        cb[...] = zero + cnt
        pltpu.sync_copy(cb, counts.at[wid, h])


@functools.cache
def _get_prep_call():
    return pl.kernel(
        _prep_body,
        out_type=(
            jax.ShapeDtypeStruct((2, NW * RCAP), jnp.int32),
            jax.ShapeDtypeStruct((2, NW * RCAP), jnp.int32),
            jax.ShapeDtypeStruct((2, NW * RCAP), jnp.int32),
            jax.ShapeDtypeStruct((NW, 2, LANE), jnp.int32),
        ),
        mesh=plsc.VectorSubcoreMesh(core_axis_name="c", subcore_axis_name="s",
                                    num_cores=NC, num_subcores=NS),
        compiler_params=pltpu.CompilerParams(use_tc_tiling_on_sc=False,
                                             needs_layout_passes=False),
        scratch_types=[
            pltpu.VMEM((PCHUNK,), jnp.int32),
            pltpu.VMEM((PCHUNK,), jnp.int32),
            pltpu.VMEM((PCHUNK,), jnp.int32),
            pltpu.VMEM((CBUF_E,), jnp.int32),
            pltpu.VMEM((CBUF_E,), jnp.int32),
            pltpu.VMEM((CBUF_E,), jnp.int32),
            pltpu.VMEM((CBUF_E,), jnp.int32),
            pltpu.VMEM((CBUF_E,), jnp.int32),
            pltpu.VMEM((CBUF_E,), jnp.int32),
            pltpu.VMEM((768,), jnp.int32),
            pltpu.VMEM((LANE,), jnp.int32),
        ],
    )


def _hop_body(table, pcols, prows, pvals, counts, out, acc,
              mvm, cntv, gvm,
              gsem0, gsem1, gsem2, msem0, msem1, ssem0, ssem1, ssem2):
    core = lax.axis_index("c")
    sid = lax.axis_index("s")
    row_base = core * HALF
    gsem = (gsem0, gsem1, gsem2)
    msem = (msem0, msem1)
    ssem = (ssem0, ssem1, ssem2)

    def _zero_row(i, carry):
        for d4 in range(D // LANE):
            gvm[0, i, pl.ds(d4 * LANE, LANE)] = jnp.zeros((LANE,), jnp.float32)
        return carry

    lax.fori_loop(0, CHUNK, _zero_row, 0)
    abase = sid * TILE_ROWS
    for t in range(TILE_ROWS // CHUNK):
        pltpu.sync_copy(gvm.at[0], acc.at[pl.ds(abase + t * CHUNK, CHUNK)])
    _zrem = TILE_ROWS % CHUNK
    if _zrem:
        pltpu.sync_copy(gvm.at[0, pl.ds(0, _zrem)],
                        acc.at[pl.ds(abase + TILE_ROWS - _zrem, _zrem)])
    plsc.subcore_barrier()

    for reg in range(2):
        w = 2 * sid + reg
        rbase = w * RBLK
        pltpu.sync_copy(counts.at[w, core], cntv)
        nstep = cntv[...][0] // CHUNK
        nblk = nstep // SUPER

        def _meta_start(g, mb):
            pltpu.async_copy(pcols.at[core, rbase + g], mvm.at[mb, 0], msem[mb])
            pltpu.async_copy(prows.at[core, rbase + g], mvm.at[mb, 1], msem[mb])
            pltpu.async_copy(pvals.at[core, rbase + g], mvm.at[mb, 2], msem[mb])

        def _meta_wait(mb):
            pltpu.make_async_copy(pcols.at[core, pl.ds(rbase, 3)], mvm.at[mb],
                                  msem[mb]).wait()

        def _gather_start(gb, mb, j):
            pltpu.async_copy(table.at[mvm.at[mb, 0, j]], gvm.at[gb], gsem[gb])

        def _gather_wait(gb, mb, j):
            pltpu.make_async_copy(table.at[pl.ds(0, CHUNK)], gvm.at[gb],
                                  gsem[gb]).wait()

        def _scatter_start(gb, mb, j):
            pltpu.async_copy(gvm.at[gb], acc.at[mvm.at[mb, 1, j]], ssem[gb],
                             add=True)

        def _scatter_wait(gb):
            pltpu.make_async_copy(gvm.at[gb], acc.at[pl.ds(0, CHUNK)],
                                  ssem[gb]).wait()

        @pl.when(nstep >= 6)
        def _():
            _meta_start(0, 0)
            _meta_start(1, 1)
            _meta_wait(0)
            _gather_start(0, 0, 0)
            _gather_start(1, 0, 1)

        def _step(i6, k):
            t = 6 * i6 + k
            gb = k % 3
            mb = (k // SUPER) % 2
            j = k % SUPER
            k2 = (k + 2) % 6
            gb2 = k2 % 3
            mb2 = (k2 // SUPER) % 2
            j2 = k2 % SUPER

            _gather_wait(gb, mb, j)

            @pl.when((t >= 1) & (t + 2 < nstep))
            def _():
                _scatter_wait(gb2)

            @pl.when(t + 2 < nstep)
            def _():
                if j2 == 0:
                    _meta_wait(mb2)
                _gather_start(gb2, mb2, j2)

            def _scale(g, carry2):
                vv = plsc.bitcast(mvm[mb, 2, j, pl.ds(g * LANE, LANE)],
                                  jnp.float32)
                for k_ in range(LANE):
                    v = vv[k_]
                    e = g * LANE + k_
                    for d4 in range(D // LANE):
                        sl = pl.ds(d4 * LANE, LANE)
                        gvm[gb, e, sl] = gvm[gb, e, sl] * v
                return carry2

            lax.fori_loop(0, CHUNK // LANE, _scale, 0)

            if k % SUPER == SUPER - 1:
                g_next = 2 * i6 + (k // SUPER) + 2

                @pl.when(g_next < nblk)
                def _():
                    _meta_start(g_next, mb)

            _scatter_start(gb, mb, j)

        def _step6(i6, carry):
            for k in range(6):
                _step(i6, k)
            return carry

        lax.fori_loop(0, nstep // 6, _step6, 0)

        @pl.when(nstep >= 6)
        def _():
            _scatter_wait(0)
            _scatter_wait(1)
            _scatter_wait(2)

    plsc.subcore_barrier()

    out_base = row_base + sid * TILE_ROWS
    pltpu.sync_copy(acc.at[pl.ds(abase, LAST_ROWS)],
                    out.at[pl.ds(out_base, LAST_ROWS)])

    @pl.when(sid < NS - 1)
    def _():
        pltpu.sync_copy(acc.at[pl.ds(abase + LAST_ROWS, TILE_ROWS - LAST_ROWS)],
                        out.at[pl.ds(out_base + LAST_ROWS, TILE_ROWS - LAST_ROWS)])


@functools.cache
def _get_hop_call():
    return pl.kernel(
        _hop_body,
        out_type=jax.ShapeDtypeStruct((N_NODES, D), jnp.float32),
        mesh=plsc.VectorSubcoreMesh(core_axis_name="c", subcore_axis_name="s",
                                    num_cores=NC, num_subcores=NS),
        compiler_params=pltpu.CompilerParams(use_tc_tiling_on_sc=False,
                                             needs_layout_passes=False),
        scratch_types=[
            pltpu.VMEM_SHARED((ACC_ROWS, D), jnp.float32),
            pltpu.VMEM((2, 3, SUPER, CHUNK), jnp.int32),
            pltpu.VMEM((LANE,), jnp.int32),
            pltpu.VMEM((3, CHUNK, D), jnp.float32),
        ] + [pltpu.SemaphoreType.DMA] * 8,
    )

U_PER_W = BATCH // NW
IDX_WORDS = 2 * U_PER_W + NEGS * U_PER_W


def _gather_body(e0, e1, e2, e3, users, pos, negs_t, s_out, p_out, n_out,
                 ivm, gvm, sem):
    core = lax.axis_index("c")
    sid = lax.axis_index("s")
    wid = sid * NC + core
    ub = wid * U_PER_W
    pltpu.sync_copy(users.at[pl.ds(ub, U_PER_W)], ivm.at[pl.ds(0, U_PER_W)])
    pltpu.sync_copy(pos.at[pl.ds(ub, U_PER_W)], ivm.at[pl.ds(U_PER_W, U_PER_W)])
    for j in range(NEGS):
        pltpu.sync_copy(negs_t.at[pl.ds(j * BATCH + ub, U_PER_W)],
                        ivm.at[pl.ds((2 + j) * U_PER_W, U_PER_W)])
    for g in range(U_PER_W // LANE, IDX_WORDS // LANE):
        sl = pl.ds(g * LANE, LANE)
        ivm[sl] = ivm[sl] + N_USERS
    for l, t in enumerate((e0, e1, e2, e3)):
        pltpu.async_copy(t.at[ivm.at[pl.ds(0, U_PER_W)]], gvm, sem).wait()
        pltpu.sync_copy(gvm, s_out.at[l, pl.ds(ub, U_PER_W)])
        pltpu.async_copy(t.at[ivm.at[pl.ds(U_PER_W, U_PER_W)]], gvm, sem).wait()
        pltpu.sync_copy(gvm, p_out.at[l, pl.ds(ub, U_PER_W)])
        for j in range(NEGS):
            pltpu.async_copy(t.at[ivm.at[pl.ds((2 + j) * U_PER_W, U_PER_W)]],
                             gvm, sem).wait()
            pltpu.sync_copy(gvm, n_out.at[l, j, pl.ds(ub, U_PER_W)])


@functools.cache
def _get_gather_call():
    return pl.kernel(
        _gather_body,
        out_type=(
            jax.ShapeDtypeStruct((4, BATCH, D), jnp.float32),
            jax.ShapeDtypeStruct((4, BATCH, D), jnp.float32),
            jax.ShapeDtypeStruct((4, NEGS, BATCH, D), jnp.float32),
        ),
        mesh=plsc.VectorSubcoreMesh(core_axis_name="c", subcore_axis_name="s",
                                    num_cores=NC, num_subcores=NS),
        compiler_params=pltpu.CompilerParams(use_tc_tiling_on_sc=False),
        scratch_types=[
            pltpu.VMEM((IDX_WORDS,), jnp.int32),
            pltpu.VMEM((U_PER_W, D), jnp.float32),
            pltpu.SemaphoreType.DMA,
        ],
    )

BB = 256
GB = BATCH // BB


def _dotT(x, w):
    return lax.dot_general(x, w, (((1,), (1,)), ((), ())),
                           preferred_element_type=jnp.float32)


def _loss_body(factor_ref, s_ref, p_ref, n_ref, wu, bu, wi, bi, wp, bp,
               wn, bn, loss_ref, reg_ref):
    factor = factor_ref[0, 0]
    u_acc = jnp.zeros((BB, D), jnp.float32)
    pos_acc = jnp.zeros((BB, D), jnp.float32)
    neg_acc = jnp.zeros((BB, D), jnp.float32)
    sel0 = jnp.zeros((BB, D), jnp.float32)
    for l in range(4):
        s_l = s_ref[l]
        p_l = p_ref[l]
        gate_p = jax.nn.sigmoid(_dotT(p_l, wi[...]) + bi[...]
                                + _dotT(s_l, wu[...]) + bu[...])
        gated_p = p_l * gate_p
        gp = _dotT(gated_p, wp[...]) + bp[...]
        best = jnp.full((BB, 1), -1e30, jnp.float32)
        bidx = jnp.zeros((BB, 1), jnp.int32)
        for j in range(NEGS):
            n_j = n_ref[l, j]
            gate_n = jax.nn.sigmoid(_dotT(n_j, wn[...]) + bn[...] + gp)
            n_sel = factor * n_j - n_j * gate_n
            sc = jnp.sum(n_sel * s_l, axis=1, keepdims=True)
            upd = sc > best
            bidx = jnp.where(upd, j, bidx)
            best = jnp.where(upd, sc, best)
        sel = jnp.zeros((BB, D), jnp.float32)
        for j in range(NEGS):
            sel = sel + jnp.where(bidx == j, n_ref[l, j], 0.0)
        u_acc = u_acc + s_l
        pos_acc = pos_acc + p_l
        neg_acc = neg_acc + sel
        if l == 0:
            sel0 = sel
            reg_blk = (jnp.sum(s_l * s_l) + jnp.sum(p_l * p_l))
    reg_blk = reg_blk + jnp.sum(sel0 * sel0)
    u_e = u_acc * 0.25
    pos_e = pos_acc * 0.25
    neg_e = neg_acc * 0.25
    d_sc = jnp.sum(u_e * neg_e, axis=1) - jnp.sum(u_e * pos_e, axis=1)
    blk_loss = jnp.sum(jnp.log(1.0 + jnp.exp(d_sc)))

    @pl.when(pl.program_id(0) == 0)
    def _():
        loss_ref[0, 0] = 0.0
        reg_ref[0, 0] = 0.0

    loss_ref[0, 0] += blk_loss
    reg_ref[0, 0] += reg_blk


_loss_call = pl.pallas_call(
    _loss_body,
    grid=(GB,),
    in_specs=[
        pl.BlockSpec(memory_space=pltpu.SMEM),
        pl.BlockSpec((4, BB, D), lambda i: (0, i, 0)),
        pl.BlockSpec((4, BB, D), lambda i: (0, i, 0)),
        pl.BlockSpec((4, NEGS, BB, D), lambda i: (0, 0, i, 0)),
        pl.BlockSpec((D, D), lambda i: (0, 0)),
        pl.BlockSpec((1, D), lambda i: (0, 0)),
        pl.BlockSpec((D, D), lambda i: (0, 0)),
        pl.BlockSpec((1, D), lambda i: (0, 0)),
        pl.BlockSpec((D, D), lambda i: (0, 0)),
        pl.BlockSpec((1, D), lambda i: (0, 0)),
        pl.BlockSpec((D, D), lambda i: (0, 0)),
        pl.BlockSpec((1, D), lambda i: (0, 0)),
    ],
    out_specs=[
        pl.BlockSpec(memory_space=pltpu.SMEM),
        pl.BlockSpec(memory_space=pltpu.SMEM),
    ],
    out_shape=[
        jax.ShapeDtypeStruct((1, 1), jnp.float32),
        jax.ShapeDtypeStruct((1, 1), jnp.float32),
    ],
)


def kernel(cur_epoch, users, pos_items, neg_items, adj_rows, adj_cols,
           adj_vals, user_embed, item_embed,
           W_user_gate, b_user_gate, W_item_gate, b_item_gate,
           W_pos_gate, b_pos_gate, W_neg_gate, b_neg_gate):
    pad = NNZ_PAD - NNZ
    rows_p = jnp.concatenate([adj_rows, jnp.zeros((pad,), jnp.int32)])
    cols_p = jnp.concatenate([adj_cols, jnp.zeros((pad,), jnp.int32)])
    vals_p = jnp.concatenate([adj_vals, jnp.zeros((pad,), jnp.float32)])
    vals_i = lax.bitcast_convert_type(vals_p, jnp.int32)
    pcols, prows, pvals, counts = _get_prep_call()(cols_p, rows_p, vals_i)
    pcols4 = pcols.reshape(2, NBLK_ALL, SUPER, CHUNK)
    prows4 = prows.reshape(2, NBLK_ALL, SUPER, CHUNK)
    pvals4 = pvals.reshape(2, NBLK_ALL, SUPER, CHUNK)
    e0 = jnp.concatenate([user_embed, item_embed], axis=0)
    hop = _get_hop_call()
    e1 = hop(e0, pcols4, prows4, pvals4, counts)
    e2 = hop(e1, pcols4, prows4, pvals4, counts)
    e3 = hop(e2, pcols4, prows4, pvals4, counts)
    negs_t = neg_items.T.reshape(-1)
    s_all, p_all, n_all = _get_gather_call()(e0, e1, e2, e3, users,
                                             pos_items, negs_t)
    factor = (1.0 - jnp.minimum(
        1.0, jnp.asarray(cur_epoch).astype(jnp.float32) / WARMUP)).reshape(1, 1)
    loss_sum, reg_sum = _loss_call(
        factor, s_all, p_all, n_all,
        W_user_gate, b_user_gate.reshape(1, D),
        W_item_gate, b_item_gate.reshape(1, D),
        W_pos_gate, b_pos_gate.reshape(1, D),
        W_neg_gate, b_neg_gate.reshape(1, D))
    mf_loss = loss_sum[0, 0] / BATCH
    emb_loss = (DECAY / (2.0 * BATCH)) * reg_sum[0, 0]
    return mf_loss + emb_loss, mf_loss, emb_loss
